# baseline (speedup 1.0000x reference)
import jax
import jax.numpy as jnp
from jax.experimental import pallas as pl
from jax.experimental.pallas import tpu as pltpu

_ROW_BLK = 512


def _top_bot_kernel(x_ref, y_ref, c_ref, q_ref, at_ref, top_ref, bot_ref):
    i = pl.program_id(0)
    z = (
        jax.lax.dot(q_ref[...], x_ref[...], preferred_element_type=jnp.float32)
        + jax.lax.dot(at_ref[...], y_ref[...], preferred_element_type=jnp.float32)
        + c_ref[...]
    )
    partial_top = jnp.sum(jnp.abs(z))
    partial_bot = jnp.sum(jnp.abs(c_ref[...]))

    @pl.when(i == 0)
    def _init():
        top_ref[0, 0] = partial_top
        bot_ref[0, 0] = partial_bot

    @pl.when(i != 0)
    def _acc():
        top_ref[0, 0] += partial_top
        bot_ref[0, 0] += partial_bot


def kernel(Q, AT, b, c, x, y):
    del b
    n = Q.shape[0]
    grid = (n // _ROW_BLK,)
    c2 = c[:, None]
    top, bot = pl.pallas_call(
        _top_bot_kernel,
        grid=grid,
        in_specs=[
            pl.BlockSpec((n, 1), lambda i: (0, 0)),
            pl.BlockSpec((n, 1), lambda i: (0, 0)),
            pl.BlockSpec((_ROW_BLK, 1), lambda i: (i, 0)),
            pl.BlockSpec((_ROW_BLK, n), lambda i: (i, 0)),
            pl.BlockSpec((_ROW_BLK, n), lambda i: (i, 0)),
        ],
        out_specs=[
            pl.BlockSpec((1, 1), lambda i: (0, 0), memory_space=pltpu.SMEM),
            pl.BlockSpec((1, 1), lambda i: (0, 0), memory_space=pltpu.SMEM),
        ],
        out_shape=[
            jax.ShapeDtypeStruct((1, 1), jnp.float32),
            jax.ShapeDtypeStruct((1, 1), jnp.float32),
        ],
    )(x, y, c2, Q, AT)
    return top[0, 0] / (10000.0 + bot[0, 0])
